# baseline (speedup 1.0000x reference)
import functools
import math

import jax
import jax.numpy as jnp
from jax import lax
from jax.experimental import pallas as pl
from jax.experimental.pallas import tpu as pltpu
from jax.experimental.pallas import tpu_sc as plsc

N = 2048
E = 32768
H = 1024
HEADS = 8
DQ = H // HEADS
SCALE = 1.0 / math.sqrt(DQ)

BLK = 256
NB = N // BLK

NC = 2
NS = 16
HALF_ROWS = N // NC
HALF_WORDS = (N * N) // NC
COPY_WORDS = HALF_WORDS // NS
EDGES_PER_TILE = E // NS
CHUNK = 128
NCHUNK = EDGES_PER_TILE // CHUNK


NPASS = 2
PASS_ROWS = HALF_ROWS // NPASS
PASS_WORDS = PASS_ROWS * N
TILE_WORDS = PASS_WORDS // NS


ROWS_PER_TILE = PASS_ROWS // NS


def _sc_gate_body(agg, subs, objs, g_out, subs_v, objs_v, idx_v, upd_v,
                  gate_sh, sem):
    c = lax.axis_index("c")
    s = lax.axis_index("s")
    ebase = s * EDGES_PER_TILE
    pltpu.sync_copy(subs.at[pl.ds(ebase, EDGES_PER_TILE)], subs_v)
    pltpu.sync_copy(objs.at[pl.ds(ebase, EDGES_PER_TILE)], objs_v)
    for v in range(CHUNK // 16):
        upd_v[pl.ds(v * 16, 16)] = jnp.full((16,), 2, jnp.int32)

    for p in range(NPASS):
        base = c * HALF_ROWS + p * PASS_ROWS
        lr0 = s * ROWS_PER_TILE
        ins = [pltpu.async_copy(agg.at[base + lr0 + r],
                                gate_sh.at[pl.ds((lr0 + r) * N, N)], sem)
               for r in range(ROWS_PER_TILE)]
        for d in ins:
            d.wait()
        plsc.subcore_barrier()
        for j in range(NCHUNK):
            for v in range(CHUNK // 16):
                sub = subs_v[pl.ds(j * CHUNK + v * 16, 16)]
                obj = objs_v[pl.ds(j * CHUNK + v * 16, 16)]
                local = (sub - base) * N + obj
                valid = (sub >= base) & (sub < base + PASS_ROWS)
                idx_v[j, pl.ds(v * 16, 16)] = jnp.where(valid, local, base)
        descs = [pltpu.async_copy(upd_v, gate_sh.at[idx_v.at[j]], sem, add=True)
                 for j in range(NCHUNK)]
        for d in descs:
            d.wait()
        plsc.subcore_barrier()
        outs = [pltpu.async_copy(gate_sh.at[pl.ds((lr0 + r) * N, N)],
                                 g_out.at[base + lr0 + r], sem)
                for r in range(ROWS_PER_TILE)]
        for d in outs:
            d.wait()


@functools.cache
def _sc_gate():
    return pl.kernel(
        _sc_gate_body,
        out_type=jax.ShapeDtypeStruct((N, N), jnp.int32),
        mesh=plsc.VectorSubcoreMesh(core_axis_name="c", subcore_axis_name="s",
                                    num_cores=NC, num_subcores=NS),
        scratch_types=[
            pltpu.VMEM((EDGES_PER_TILE,), jnp.int32),
            pltpu.VMEM((EDGES_PER_TILE,), jnp.int32),
            pltpu.VMEM((NCHUNK, CHUNK), jnp.int32),
            pltpu.VMEM((CHUNK,), jnp.int32),
            pltpu.VMEM_SHARED((PASS_WORDS,), jnp.int32),
            pltpu.SemaphoreType.DMA,
        ],
    )


def _proj_body(x_ref, wq_ref, bq_ref, wk_ref, bk_ref, q_ref, k_ref):
    x = x_ref[...]
    q_ref[...] = jnp.dot(x, wq_ref[...], preferred_element_type=jnp.float32) + bq_ref[...]
    k_ref[...] = jnp.dot(x, wk_ref[...], preferred_element_type=jnp.float32) + bk_ref[...]


def _attn_body(q_ref, k_ref, g_ref, x_ref, o_ref):
    i = pl.program_id(0)
    g = g_ref[...]
    agg_ok = (g & 1) == 1
    edge = g >= 2
    row_ids = i * BLK + lax.broadcasted_iota(jnp.int32, (BLK, N), 0)
    col_ids = lax.broadcasted_iota(jnp.int32, (BLK, N), 1)
    diag = row_ids == col_ids
    gmul = jnp.where(agg_ok & edge & (~diag), SCALE, 0.0)
    gadd = jnp.where(diag, 1e-7, jnp.where(agg_ok, 0.0, -1e9))
    acc = jnp.zeros((BLK, N), jnp.float32)
    for h in range(HEADS):
        qh = q_ref[:, h * DQ:(h + 1) * DQ]
        kh = k_ref[:, h * DQ:(h + 1) * DQ]
        s = lax.dot_general(qh, kh, (((1,), (1,)), ((), ())),
                            preferred_element_type=jnp.float32,
                            precision=lax.Precision.HIGHEST)
        a = s * gmul + gadd
        m = jnp.max(a, axis=1, keepdims=True)
        e = jnp.exp(a - m)
        inv = 1.0 / jnp.sum(e, axis=1, keepdims=True)
        acc = acc + e * inv
    p = acc * (1.0 / HEADS)
    o_ref[...] = jnp.dot(p, x_ref[...], preferred_element_type=jnp.float32)


def kernel(inst_feature, aggregator_matrix, rel_pair_index, Wq, bq, Wk, bk):
    g = _sc_gate()(aggregator_matrix,
                   rel_pair_index[:, 0], rel_pair_index[:, 1])

    q, k = pl.pallas_call(
        _proj_body,
        grid=(NB,),
        in_specs=[
            pl.BlockSpec((BLK, H), lambda i: (i, 0)),
            pl.BlockSpec((H, H), lambda i: (0, 0)),
            pl.BlockSpec((1, H), lambda i: (0, 0)),
            pl.BlockSpec((H, H), lambda i: (0, 0)),
            pl.BlockSpec((1, H), lambda i: (0, 0)),
        ],
        out_specs=[
            pl.BlockSpec((BLK, H), lambda i: (i, 0)),
            pl.BlockSpec((BLK, H), lambda i: (i, 0)),
        ],
        out_shape=[
            jax.ShapeDtypeStruct((N, H), jnp.float32),
            jax.ShapeDtypeStruct((N, H), jnp.float32),
        ],
    )(inst_feature, Wq, bq.reshape(1, H), Wk, bk.reshape(1, H))

    out = pl.pallas_call(
        _attn_body,
        grid=(NB,),
        in_specs=[
            pl.BlockSpec((BLK, H), lambda i: (i, 0)),
            pl.BlockSpec((N, H), lambda i: (0, 0)),
            pl.BlockSpec((BLK, N), lambda i: (i, 0)),
            pl.BlockSpec((N, H), lambda i: (0, 0)),
        ],
        out_specs=pl.BlockSpec((BLK, H), lambda i: (i, 0)),
        out_shape=jax.ShapeDtypeStruct((N, H), jnp.float32),
    )(q, k, g, inst_feature)
    return out


# speedup vs baseline: 1.4532x; 1.4532x over previous
import functools
import math

import jax
import jax.numpy as jnp
from jax import lax
from jax.experimental import pallas as pl
from jax.experimental.pallas import tpu as pltpu
from jax.experimental.pallas import tpu_sc as plsc

N = 2048
E = 32768
H = 1024
HEADS = 8
DQ = H // HEADS
SCALE = 1.0 / math.sqrt(DQ)

BLK = 256
NB = N // BLK

NC = 2
NS = 16
HALF_ROWS = N // NC
HALF_WORDS = (N * N) // NC
COPY_WORDS = HALF_WORDS // NS
EDGES_PER_TILE = E // NS
CHUNK = 128
NCHUNK = EDGES_PER_TILE // CHUNK


NPASS = 2
PASS_ROWS = HALF_ROWS // NPASS
PASS_WORDS = PASS_ROWS * N
TILE_WORDS = PASS_WORDS // NS


ROWS_PER_TILE = PASS_ROWS // NS


def _sc_gate_body(agg, subs, objs, g_out, subs_v, objs_v, idx_v, upd_v,
                  gate_sh, sem):
    c = lax.axis_index("c")
    s = lax.axis_index("s")
    ebase = s * EDGES_PER_TILE
    pltpu.sync_copy(subs.at[pl.ds(ebase, EDGES_PER_TILE)], subs_v)
    pltpu.sync_copy(objs.at[pl.ds(ebase, EDGES_PER_TILE)], objs_v)
    for v in range(CHUNK // 16):
        upd_v[pl.ds(v * 16, 16)] = jnp.full((16,), 2, jnp.int32)

    for p in range(NPASS):
        base = c * HALF_ROWS + p * PASS_ROWS
        lr0 = s * ROWS_PER_TILE
        ins = [pltpu.async_copy(agg.at[base + lr0 + r],
                                gate_sh.at[pl.ds((lr0 + r) * N, N)], sem)
               for r in range(ROWS_PER_TILE)]
        for d in ins:
            d.wait()
        plsc.subcore_barrier()
        for j in range(NCHUNK):
            for v in range(CHUNK // 16):
                sub = subs_v[pl.ds(j * CHUNK + v * 16, 16)]
                obj = objs_v[pl.ds(j * CHUNK + v * 16, 16)]
                local = (sub - base) * N + obj
                valid = (sub >= base) & (sub < base + PASS_ROWS)
                idx_v[j, pl.ds(v * 16, 16)] = jnp.where(valid, local, base)
        descs = [pltpu.async_copy(upd_v, gate_sh.at[idx_v.at[j]], sem, add=True)
                 for j in range(NCHUNK)]
        for d in descs:
            d.wait()
        plsc.subcore_barrier()
        outs = [pltpu.async_copy(gate_sh.at[pl.ds((lr0 + r) * N, N)],
                                 g_out.at[base + lr0 + r], sem)
                for r in range(ROWS_PER_TILE)]
        for d in outs:
            d.wait()


@functools.cache
def _sc_gate():
    return pl.kernel(
        _sc_gate_body,
        out_type=jax.ShapeDtypeStruct((N, N), jnp.int32),
        mesh=plsc.VectorSubcoreMesh(core_axis_name="c", subcore_axis_name="s",
                                    num_cores=NC, num_subcores=NS),
        scratch_types=[
            pltpu.VMEM((EDGES_PER_TILE,), jnp.int32),
            pltpu.VMEM((EDGES_PER_TILE,), jnp.int32),
            pltpu.VMEM((NCHUNK, CHUNK), jnp.int32),
            pltpu.VMEM((CHUNK,), jnp.int32),
            pltpu.VMEM_SHARED((PASS_WORDS,), jnp.int32),
            pltpu.SemaphoreType.DMA,
        ],
    )


def _proj_body(x_ref, wq_ref, bq_ref, wk_ref, bk_ref, q_ref, k_ref):
    x = x_ref[...]
    q_ref[...] = jnp.dot(x, wq_ref[...], preferred_element_type=jnp.float32) + bq_ref[...]
    k_ref[...] = jnp.dot(x, wk_ref[...], preferred_element_type=jnp.float32) + bk_ref[...]


def _attn_body(q_ref, k_ref, g_ref, x_ref, o_ref):
    i = pl.program_id(0)
    g = g_ref[...]
    agg_ok = (g & 1) == 1
    edge = g >= 2
    row_ids = i * BLK + lax.broadcasted_iota(jnp.int32, (BLK, N), 0)
    col_ids = lax.broadcasted_iota(jnp.int32, (BLK, N), 1)
    diag = row_ids == col_ids
    gmul = jnp.where(agg_ok & edge & (~diag), SCALE, 0.0)
    gadd = jnp.where(diag, 1e-7, jnp.where(agg_ok, 0.0, -1e9))
    acc = jnp.zeros((BLK, N), jnp.float32)
    for h in range(HEADS):
        qh = q_ref[:, h * DQ:(h + 1) * DQ]
        kh = k_ref[:, h * DQ:(h + 1) * DQ]
        s = lax.dot_general(qh, kh, (((1,), (1,)), ((), ())),
                            preferred_element_type=jnp.float32)
        a = s * gmul + gadd
        m = jnp.max(a, axis=1, keepdims=True)
        e = jnp.exp(a - m)
        den = jnp.sum(e, axis=1, keepdims=True)
        inv = 1.0 / den
        inv = inv * (2.0 - den * inv)
        acc = acc + e * inv
    p = acc * (1.0 / HEADS)
    o_ref[...] = jnp.dot(p, x_ref[...], preferred_element_type=jnp.float32)


def kernel(inst_feature, aggregator_matrix, rel_pair_index, Wq, bq, Wk, bk):
    g = _sc_gate()(aggregator_matrix,
                   rel_pair_index[:, 0], rel_pair_index[:, 1])

    q, k = pl.pallas_call(
        _proj_body,
        grid=(NB,),
        in_specs=[
            pl.BlockSpec((BLK, H), lambda i: (i, 0)),
            pl.BlockSpec((H, H), lambda i: (0, 0)),
            pl.BlockSpec((1, H), lambda i: (0, 0)),
            pl.BlockSpec((H, H), lambda i: (0, 0)),
            pl.BlockSpec((1, H), lambda i: (0, 0)),
        ],
        out_specs=[
            pl.BlockSpec((BLK, H), lambda i: (i, 0)),
            pl.BlockSpec((BLK, H), lambda i: (i, 0)),
        ],
        out_shape=[
            jax.ShapeDtypeStruct((N, H), jnp.float32),
            jax.ShapeDtypeStruct((N, H), jnp.float32),
        ],
    )(inst_feature, Wq, bq.reshape(1, H), Wk, bk.reshape(1, H))

    out = pl.pallas_call(
        _attn_body,
        grid=(NB,),
        in_specs=[
            pl.BlockSpec((BLK, H), lambda i: (i, 0)),
            pl.BlockSpec((N, H), lambda i: (0, 0)),
            pl.BlockSpec((BLK, N), lambda i: (i, 0)),
            pl.BlockSpec((N, H), lambda i: (0, 0)),
        ],
        out_specs=pl.BlockSpec((BLK, H), lambda i: (i, 0)),
        out_shape=jax.ShapeDtypeStruct((N, H), jnp.float32),
    )(q, k, g, inst_feature)
    return out


# speedup vs baseline: 1.6088x; 1.1071x over previous
import functools
import math

import jax
import jax.numpy as jnp
from jax import lax
from jax.experimental import pallas as pl
from jax.experimental.pallas import tpu as pltpu
from jax.experimental.pallas import tpu_sc as plsc

N = 2048
E = 32768
H = 1024
HEADS = 8
DQ = H // HEADS
SCALE = 1.0 / math.sqrt(DQ)

BLK = 256
NB = N // BLK

NC = 2
NS = 16
HALF_ROWS = N // NC
HALF_WORDS = (N * N) // NC
COPY_WORDS = HALF_WORDS // NS
EDGES_PER_TILE = E // NS
CHUNK = 128
NCHUNK = EDGES_PER_TILE // CHUNK


NPASS = 2
PASS_ROWS = HALF_ROWS // NPASS
PASS_WORDS = PASS_ROWS * N
TILE_WORDS = PASS_WORDS // NS


ROWS_PER_TILE = PASS_ROWS // NS


def _sc_gate_body(agg, subs, objs, g_out, subs_v, objs_v, idx_v, upd_v,
                  gate_sh, sem):
    c = lax.axis_index("c")
    s = lax.axis_index("s")
    ebase = s * EDGES_PER_TILE
    pltpu.sync_copy(subs.at[pl.ds(ebase, EDGES_PER_TILE)], subs_v)
    pltpu.sync_copy(objs.at[pl.ds(ebase, EDGES_PER_TILE)], objs_v)
    for v in range(CHUNK // 16):
        upd_v[pl.ds(v * 16, 16)] = jnp.full((16,), 2, jnp.int32)

    for p in range(NPASS):
        base = c * HALF_ROWS + p * PASS_ROWS
        lr0 = s * ROWS_PER_TILE
        ins = [pltpu.async_copy(agg.at[base + lr0 + r],
                                gate_sh.at[pl.ds((lr0 + r) * N, N)], sem)
               for r in range(ROWS_PER_TILE)]
        for d in ins:
            d.wait()
        plsc.subcore_barrier()
        for j in range(NCHUNK):
            for v in range(CHUNK // 16):
                sub = subs_v[pl.ds(j * CHUNK + v * 16, 16)]
                obj = objs_v[pl.ds(j * CHUNK + v * 16, 16)]
                local = (sub - base) * N + obj
                valid = (sub >= base) & (sub < base + PASS_ROWS)
                idx_v[j, pl.ds(v * 16, 16)] = jnp.where(valid, local, base)
        descs = [pltpu.async_copy(upd_v, gate_sh.at[idx_v.at[j]], sem, add=True)
                 for j in range(NCHUNK)]
        for d in descs:
            d.wait()
        plsc.subcore_barrier()
        outs = [pltpu.async_copy(gate_sh.at[pl.ds((lr0 + r) * N, N)],
                                 g_out.at[base + lr0 + r], sem)
                for r in range(ROWS_PER_TILE)]
        for d in outs:
            d.wait()


@functools.cache
def _sc_gate():
    return pl.kernel(
        _sc_gate_body,
        out_type=jax.ShapeDtypeStruct((N, N), jnp.int32),
        mesh=plsc.VectorSubcoreMesh(core_axis_name="c", subcore_axis_name="s",
                                    num_cores=NC, num_subcores=NS),
        scratch_types=[
            pltpu.VMEM((EDGES_PER_TILE,), jnp.int32),
            pltpu.VMEM((EDGES_PER_TILE,), jnp.int32),
            pltpu.VMEM((NCHUNK, CHUNK), jnp.int32),
            pltpu.VMEM((CHUNK,), jnp.int32),
            pltpu.VMEM_SHARED((PASS_WORDS,), jnp.int32),
            pltpu.SemaphoreType.DMA,
        ],
    )


def _proj_body(x_ref, wq_ref, bq_ref, wk_ref, bk_ref, q_ref, k_ref):
    x = x_ref[...]
    q_ref[...] = jnp.dot(x, wq_ref[...], preferred_element_type=jnp.float32) + bq_ref[...]
    k_ref[...] = jnp.dot(x, wk_ref[...], preferred_element_type=jnp.float32) + bk_ref[...]


def _attn_body(q_ref, k_ref, g_ref, x_ref, o_ref):
    i = pl.program_id(0)
    g = g_ref[...]
    agg_ok = (g & 1) == 1
    edge = g >= 2
    row_ids = i * BLK + lax.broadcasted_iota(jnp.int32, (BLK, N), 0)
    col_ids = lax.broadcasted_iota(jnp.int32, (BLK, N), 1)
    diag = row_ids == col_ids
    gmul = jnp.where(agg_ok & edge & (~diag), SCALE, 0.0)
    gadd = jnp.where(diag, 1e-7, jnp.where(agg_ok, 0.0, -1e9))
    acc = jnp.zeros((BLK, N), jnp.float32)
    for h in range(HEADS):
        qh = q_ref[:, h * DQ:(h + 1) * DQ]
        kh = k_ref[:, h * DQ:(h + 1) * DQ]
        s = lax.dot_general(qh, kh, (((1,), (1,)), ((), ())),
                            preferred_element_type=jnp.float32)
        e = jnp.exp(s * gmul + gadd)
        den = jnp.sum(e, axis=1, keepdims=True)
        inv = 1.0 / den
        inv = inv * (2.0 - den * inv)
        acc = acc + e * inv
    p = acc * (1.0 / HEADS)
    o_ref[...] = jnp.dot(p, x_ref[...], preferred_element_type=jnp.float32)


def kernel(inst_feature, aggregator_matrix, rel_pair_index, Wq, bq, Wk, bk):
    g = _sc_gate()(aggregator_matrix,
                   rel_pair_index[:, 0], rel_pair_index[:, 1])

    q, k = pl.pallas_call(
        _proj_body,
        grid=(NB,),
        in_specs=[
            pl.BlockSpec((BLK, H), lambda i: (i, 0)),
            pl.BlockSpec((H, H), lambda i: (0, 0)),
            pl.BlockSpec((1, H), lambda i: (0, 0)),
            pl.BlockSpec((H, H), lambda i: (0, 0)),
            pl.BlockSpec((1, H), lambda i: (0, 0)),
        ],
        out_specs=[
            pl.BlockSpec((BLK, H), lambda i: (i, 0)),
            pl.BlockSpec((BLK, H), lambda i: (i, 0)),
        ],
        out_shape=[
            jax.ShapeDtypeStruct((N, H), jnp.float32),
            jax.ShapeDtypeStruct((N, H), jnp.float32),
        ],
    )(inst_feature, Wq, bq.reshape(1, H), Wk, bk.reshape(1, H))

    out = pl.pallas_call(
        _attn_body,
        grid=(NB,),
        in_specs=[
            pl.BlockSpec((BLK, H), lambda i: (i, 0)),
            pl.BlockSpec((N, H), lambda i: (0, 0)),
            pl.BlockSpec((BLK, N), lambda i: (i, 0)),
            pl.BlockSpec((N, H), lambda i: (0, 0)),
        ],
        out_specs=pl.BlockSpec((BLK, H), lambda i: (i, 0)),
        out_shape=jax.ShapeDtypeStruct((N, H), jnp.float32),
    )(q, k, g, inst_feature)
    return out


# speedup vs baseline: 1.7460x; 1.0853x over previous
import functools
import math

import jax
import jax.numpy as jnp
from jax import lax
from jax.experimental import pallas as pl
from jax.experimental.pallas import tpu as pltpu
from jax.experimental.pallas import tpu_sc as plsc

N = 2048
E = 32768
H = 1024
HEADS = 8
DQ = H // HEADS
SCALE = 1.0 / math.sqrt(DQ)

BLK = 256
NB = N // BLK

NC = 2
NS = 16
HALF_ROWS = N // NC
HALF_WORDS = (N * N) // NC
COPY_WORDS = HALF_WORDS // NS
EDGES_PER_TILE = E // NS
CHUNK = 128
NCHUNK = EDGES_PER_TILE // CHUNK


NPASS = 2
PASS_ROWS = HALF_ROWS // NPASS
PASS_WORDS = PASS_ROWS * N
TILE_WORDS = PASS_WORDS // NS


ROWS_PER_TILE = PASS_ROWS // NS


def _sc_gate_body(agg, subs, objs, g_out, subs_v, objs_v, idx_v, upd_v,
                  gate_sh, sem):
    c = lax.axis_index("c")
    s = lax.axis_index("s")
    ebase = s * EDGES_PER_TILE
    pltpu.sync_copy(subs.at[pl.ds(ebase, EDGES_PER_TILE)], subs_v)
    pltpu.sync_copy(objs.at[pl.ds(ebase, EDGES_PER_TILE)], objs_v)
    for v in range(CHUNK // 16):
        upd_v[pl.ds(v * 16, 16)] = jnp.full((16,), 2, jnp.int32)

    for p in range(NPASS):
        base = c * HALF_ROWS + p * PASS_ROWS
        lr0 = s * ROWS_PER_TILE
        ins = [pltpu.async_copy(agg.at[base + lr0 + r],
                                gate_sh.at[pl.ds((lr0 + r) * N, N)], sem)
               for r in range(ROWS_PER_TILE)]
        for d in ins:
            d.wait()
        plsc.subcore_barrier()
        for j in range(NCHUNK):
            for v in range(CHUNK // 16):
                sub = subs_v[pl.ds(j * CHUNK + v * 16, 16)]
                obj = objs_v[pl.ds(j * CHUNK + v * 16, 16)]
                local = (sub - base) * N + obj
                valid = (sub >= base) & (sub < base + PASS_ROWS)
                tr = sub & (PASS_ROWS - 1)
                trash = tr * (N + 1) + base
                idx_v[j, pl.ds(v * 16, 16)] = jnp.where(valid, local, trash)
        descs = [pltpu.async_copy(upd_v, gate_sh.at[idx_v.at[j]], sem, add=True)
                 for j in range(NCHUNK)]
        for d in descs:
            d.wait()
        plsc.subcore_barrier()
        outs = [pltpu.async_copy(gate_sh.at[pl.ds((lr0 + r) * N, N)],
                                 g_out.at[base + lr0 + r], sem)
                for r in range(ROWS_PER_TILE)]
        for d in outs:
            d.wait()


@functools.cache
def _sc_gate():
    return pl.kernel(
        _sc_gate_body,
        out_type=jax.ShapeDtypeStruct((N, N), jnp.int32),
        mesh=plsc.VectorSubcoreMesh(core_axis_name="c", subcore_axis_name="s",
                                    num_cores=NC, num_subcores=NS),
        scratch_types=[
            pltpu.VMEM((EDGES_PER_TILE,), jnp.int32),
            pltpu.VMEM((EDGES_PER_TILE,), jnp.int32),
            pltpu.VMEM((NCHUNK, CHUNK), jnp.int32),
            pltpu.VMEM((CHUNK,), jnp.int32),
            pltpu.VMEM_SHARED((PASS_WORDS,), jnp.int32),
            pltpu.SemaphoreType.DMA,
        ],
    )


def _proj_body(x_ref, wq_ref, bq_ref, wk_ref, bk_ref, q_ref, k_ref):
    x = x_ref[...]
    q_ref[...] = jnp.dot(x, wq_ref[...], preferred_element_type=jnp.float32) + bq_ref[...]
    k_ref[...] = jnp.dot(x, wk_ref[...], preferred_element_type=jnp.float32) + bk_ref[...]


def _attn_body(q_ref, k_ref, g_ref, x_ref, o_ref):
    i = pl.program_id(0)
    g = g_ref[...]
    agg_ok = (g & 1) == 1
    edge = g >= 2
    row_ids = i * BLK + lax.broadcasted_iota(jnp.int32, (BLK, N), 0)
    col_ids = lax.broadcasted_iota(jnp.int32, (BLK, N), 1)
    diag = row_ids == col_ids
    gmul = jnp.where(agg_ok & edge & (~diag), SCALE, 0.0)
    gadd = jnp.where(diag, 1e-7, jnp.where(agg_ok, 0.0, -1e9))
    acc = jnp.zeros((BLK, N), jnp.float32)
    for h in range(HEADS):
        qh = q_ref[:, h * DQ:(h + 1) * DQ]
        kh = k_ref[:, h * DQ:(h + 1) * DQ]
        s = lax.dot_general(qh, kh, (((1,), (1,)), ((), ())),
                            preferred_element_type=jnp.float32)
        a = s * gmul + gadd
        m = jnp.max(a, axis=1, keepdims=True)
        e = jnp.exp(a - m)
        den = jnp.sum(e, axis=1, keepdims=True)
        inv = 1.0 / den
        inv = inv * (2.0 - den * inv)
        acc = acc + e * inv
    p = acc * (1.0 / HEADS)
    o_ref[...] = jnp.dot(p, x_ref[...], preferred_element_type=jnp.float32)


def kernel(inst_feature, aggregator_matrix, rel_pair_index, Wq, bq, Wk, bk):
    g = _sc_gate()(aggregator_matrix,
                   rel_pair_index[:, 0], rel_pair_index[:, 1])

    q, k = pl.pallas_call(
        _proj_body,
        grid=(NB,),
        in_specs=[
            pl.BlockSpec((BLK, H), lambda i: (i, 0)),
            pl.BlockSpec((H, H), lambda i: (0, 0)),
            pl.BlockSpec((1, H), lambda i: (0, 0)),
            pl.BlockSpec((H, H), lambda i: (0, 0)),
            pl.BlockSpec((1, H), lambda i: (0, 0)),
        ],
        out_specs=[
            pl.BlockSpec((BLK, H), lambda i: (i, 0)),
            pl.BlockSpec((BLK, H), lambda i: (i, 0)),
        ],
        out_shape=[
            jax.ShapeDtypeStruct((N, H), jnp.float32),
            jax.ShapeDtypeStruct((N, H), jnp.float32),
        ],
    )(inst_feature, Wq, bq.reshape(1, H), Wk, bk.reshape(1, H))

    out = pl.pallas_call(
        _attn_body,
        grid=(NB,),
        in_specs=[
            pl.BlockSpec((BLK, H), lambda i: (i, 0)),
            pl.BlockSpec((N, H), lambda i: (0, 0)),
            pl.BlockSpec((BLK, N), lambda i: (i, 0)),
            pl.BlockSpec((N, H), lambda i: (0, 0)),
        ],
        out_specs=pl.BlockSpec((BLK, H), lambda i: (i, 0)),
        out_shape=jax.ShapeDtypeStruct((N, H), jnp.float32),
    )(q, k, g, inst_feature)
    return out


# speedup vs baseline: 1.7505x; 1.0026x over previous
import functools
import math

import jax
import jax.numpy as jnp
from jax import lax
from jax.experimental import pallas as pl
from jax.experimental.pallas import tpu as pltpu
from jax.experimental.pallas import tpu_sc as plsc

N = 2048
E = 32768
H = 1024
HEADS = 8
DQ = H // HEADS
SCALE = 1.0 / math.sqrt(DQ)

BLK = 256
NB = N // BLK

NC = 2
NS = 16
HALF_ROWS = N // NC
HALF_WORDS = (N * N) // NC
COPY_WORDS = HALF_WORDS // NS
EDGES_PER_TILE = E // NS
CHUNK = 128
NCHUNK = EDGES_PER_TILE // CHUNK


NPASS = 2
PASS_ROWS = HALF_ROWS // NPASS
PASS_WORDS = PASS_ROWS * N
TILE_WORDS = PASS_WORDS // NS


ROWS_PER_TILE = PASS_ROWS // NS


def _sc_gate_body(agg, subs, objs, g_out, subs_v, objs_v, idx_v, upd_v,
                  gate_sh, sem):
    c = lax.axis_index("c")
    s = lax.axis_index("s")
    ebase = s * EDGES_PER_TILE
    pltpu.sync_copy(subs.at[pl.ds(ebase, EDGES_PER_TILE)], subs_v)
    pltpu.sync_copy(objs.at[pl.ds(ebase, EDGES_PER_TILE)], objs_v)
    for v in range(CHUNK // 16):
        upd_v[pl.ds(v * 16, 16)] = jnp.full((16,), 2, jnp.int32)

    for p in range(NPASS):
        base = c * HALF_ROWS + p * PASS_ROWS
        lr0 = s * ROWS_PER_TILE
        ins = [pltpu.async_copy(agg.at[base + lr0 + r],
                                gate_sh.at[pl.ds((lr0 + r) * N, N)], sem)
               for r in range(ROWS_PER_TILE)]
        for d in ins:
            d.wait()
        plsc.subcore_barrier()
        for j in range(NCHUNK):
            for v in range(CHUNK // 16):
                sub = subs_v[pl.ds(j * CHUNK + v * 16, 16)]
                obj = objs_v[pl.ds(j * CHUNK + v * 16, 16)]
                local = (sub - base) * N + obj
                valid = (sub >= base) & (sub < base + PASS_ROWS)
                tr = sub & (PASS_ROWS - 1)
                trash = tr * (N + 1) + base
                idx_v[j, pl.ds(v * 16, 16)] = jnp.where(valid, local, trash)
        descs = [pltpu.async_copy(upd_v, gate_sh.at[idx_v.at[j]], sem, add=True)
                 for j in range(NCHUNK)]
        for d in descs:
            d.wait()
        plsc.subcore_barrier()
        outs = [pltpu.async_copy(gate_sh.at[pl.ds((lr0 + r) * N, N)],
                                 g_out.at[base + lr0 + r], sem)
                for r in range(ROWS_PER_TILE)]
        for d in outs:
            d.wait()


@functools.cache
def _sc_gate():
    return pl.kernel(
        _sc_gate_body,
        out_type=jax.ShapeDtypeStruct((N, N), jnp.int32),
        mesh=plsc.VectorSubcoreMesh(core_axis_name="c", subcore_axis_name="s",
                                    num_cores=NC, num_subcores=NS),
        scratch_types=[
            pltpu.VMEM((EDGES_PER_TILE,), jnp.int32),
            pltpu.VMEM((EDGES_PER_TILE,), jnp.int32),
            pltpu.VMEM((NCHUNK, CHUNK), jnp.int32),
            pltpu.VMEM((CHUNK,), jnp.int32),
            pltpu.VMEM_SHARED((PASS_WORDS,), jnp.int32),
            pltpu.SemaphoreType.DMA,
        ],
    )


def _proj_body(x_ref, wq_ref, bq_ref, wk_ref, bk_ref, q_ref, k_ref):
    x = x_ref[...]
    q_ref[...] = jnp.dot(x, wq_ref[...], preferred_element_type=jnp.float32) + bq_ref[...]
    k_ref[...] = jnp.dot(x, wk_ref[...], preferred_element_type=jnp.float32) + bk_ref[...]


def _attn_body(q_ref, k_ref, g_ref, x_ref, o_ref):
    i = pl.program_id(0)
    g = g_ref[...]
    agg_ok = (g & 1) == 1
    edge = g >= 2
    row_ids = i * BLK + lax.broadcasted_iota(jnp.int32, (BLK, N), 0)
    col_ids = lax.broadcasted_iota(jnp.int32, (BLK, N), 1)
    diag = row_ids == col_ids
    gmul = jnp.where(agg_ok & edge & (~diag), SCALE, 0.0)
    gadd = jnp.where(diag, 1e-7, jnp.where(agg_ok, 0.0, -1e9))
    acc = jnp.zeros((BLK, N), jnp.float32)
    for h in range(HEADS):
        qh = q_ref[:, h * DQ:(h + 1) * DQ]
        kh = k_ref[:, h * DQ:(h + 1) * DQ]
        s = lax.dot_general(qh, kh, (((1,), (1,)), ((), ())),
                            preferred_element_type=jnp.float32)
        a = s * gmul + gadd
        m = jnp.max(a, axis=1, keepdims=True)
        e = jnp.exp(a - m)
        den = jnp.sum(e, axis=1, keepdims=True)
        inv = 1.0 / den
        inv = inv * (2.0 - den * inv)
        acc = acc + e * inv
    o_ref[...] = jnp.dot(acc, x_ref[...],
                         preferred_element_type=jnp.float32) * (1.0 / HEADS)


def kernel(inst_feature, aggregator_matrix, rel_pair_index, Wq, bq, Wk, bk):
    g = _sc_gate()(aggregator_matrix,
                   rel_pair_index[:, 0], rel_pair_index[:, 1])

    q, k = pl.pallas_call(
        _proj_body,
        grid=(NB,),
        in_specs=[
            pl.BlockSpec((BLK, H), lambda i: (i, 0)),
            pl.BlockSpec((H, H), lambda i: (0, 0)),
            pl.BlockSpec((1, H), lambda i: (0, 0)),
            pl.BlockSpec((H, H), lambda i: (0, 0)),
            pl.BlockSpec((1, H), lambda i: (0, 0)),
        ],
        out_specs=[
            pl.BlockSpec((BLK, H), lambda i: (i, 0)),
            pl.BlockSpec((BLK, H), lambda i: (i, 0)),
        ],
        out_shape=[
            jax.ShapeDtypeStruct((N, H), jnp.float32),
            jax.ShapeDtypeStruct((N, H), jnp.float32),
        ],
    )(inst_feature, Wq, bq.reshape(1, H), Wk, bk.reshape(1, H))

    out = pl.pallas_call(
        _attn_body,
        grid=(NB,),
        in_specs=[
            pl.BlockSpec((BLK, H), lambda i: (i, 0)),
            pl.BlockSpec((N, H), lambda i: (0, 0)),
            pl.BlockSpec((BLK, N), lambda i: (i, 0)),
            pl.BlockSpec((N, H), lambda i: (0, 0)),
        ],
        out_specs=pl.BlockSpec((BLK, H), lambda i: (i, 0)),
        out_shape=jax.ShapeDtypeStruct((N, H), jnp.float32),
    )(q, k, g, inst_feature)
    return out
